# baseline (speedup 1.0000x reference)
import jax
import jax.numpy as jnp
from jax import lax
from jax.experimental import pallas as pl
from jax.experimental.pallas import tpu as pltpu
from jax.experimental.pallas import tpu_sc as plsc

N_NODES = 10000
D = 128
N_EDGES = 320000

NC = 2
NS = 16
NW = NC * NS
EPW = N_EDGES // NW
CHUNK = 80
NCHUNKS = EPW // CHUNK
LANES = 16
GROUPS = CHUNK // LANES
NBUF = 4
TSTRIDE = LANES + 1


def _body(x_req_hbm, x_code_hbm, src_hbm, dst_hbm, out_hbm,
          src_v, dst_v,
          req0, req1, req2, req3, code0, code1, code2, code3,
          out_v, tr_v, sem0, sem1, sem2, sem3):
    reqs = (req0, req1, req2, req3)
    codes = (code0, code1, code2, code3)
    sems = (sem0, sem1, sem2, sem3)

    sid = lax.axis_index("s")
    wid = sid * NC + lax.axis_index("c")
    wbase = wid * EPW

    pltpu.sync_copy(src_hbm.at[pl.ds(wbase, EPW)], src_v)
    pltpu.sync_copy(dst_hbm.at[pl.ds(wbase, EPW)], dst_v)

    def fire(c, b):
        off = c * CHUNK
        pltpu.async_copy(x_req_hbm.at[src_v.at[pl.ds(off, CHUNK)]],
                         reqs[b], sems[b])
        pltpu.async_copy(x_code_hbm.at[dst_v.at[pl.ds(off, CHUNK)]],
                         codes[b], sems[b])

    def wait_rows(b):
        pltpu.make_async_copy(x_req_hbm.at[pl.ds(0, CHUNK)], reqs[b],
                              sems[b]).wait()
        pltpu.make_async_copy(x_code_hbm.at[pl.ds(0, CHUNK)], codes[b],
                              sems[b]).wait()

    lane_ids = lax.iota(jnp.int32, LANES)
    col_base = lane_ids * TSTRIDE

    def compute(c, b):
        req_v, code_v = reqs[b], codes[b]
        obase = c * CHUNK

        @pl.loop(0, GROUPS)
        def _(g):
            gbase = g * LANES

            @plsc.parallel_loop(0, LANES, unroll=2)
            def _(e):
                edge = gbase + e
                ps = [req_v[edge, pl.ds(k * LANES, LANES)]
                      * code_v[edge, pl.ds(k * LANES, LANES)]
                      for k in range(D // LANES)]
                while len(ps) > 1:
                    ps = [ps[j] + ps[j + 1] for j in range(0, len(ps), 2)]
                plsc.store_scatter(tr_v, [col_base + e], ps[0])

            rows = [tr_v[pl.ds(l * TSTRIDE, LANES)] for l in range(LANES)]
            while len(rows) > 1:
                rows = [rows[j] + rows[j + 1] for j in range(0, len(rows), 2)]
            sig = 1.0 / (1.0 + jnp.exp(-rows[0]))
            out_v[pl.ds(obase + gbase, LANES)] = sig

    for b in range(NBUF - 1):
        fire(b, b)

    @pl.loop(0, NCHUNKS - 1, step=NBUF)
    def _(i):
        for b in range(NBUF):
            c = i + b
            wait_rows(b)
            nxt = c + NBUF - 1

            @pl.when(nxt < NCHUNKS)
            def _():
                fire(nxt, (b + NBUF - 1) % NBUF)

            compute(c, b)

    wait_rows(0)
    compute(NCHUNKS - 1, 0)

    pltpu.sync_copy(out_v, out_hbm.at[pl.ds(wbase, EPW)])


def _make_classifier():
    mesh = plsc.VectorSubcoreMesh(core_axis_name="c", subcore_axis_name="s",
                                  num_cores=NC, num_subcores=NS)
    row_t = pltpu.VMEM((CHUNK, D), jnp.float32)
    return pl.kernel(
        _body,
        out_type=jax.ShapeDtypeStruct((N_EDGES,), jnp.float32),
        mesh=mesh,
        scratch_types=[
            pltpu.VMEM((EPW,), jnp.int32),
            pltpu.VMEM((EPW,), jnp.int32),
            row_t, row_t, row_t, row_t, row_t, row_t, row_t, row_t,
            pltpu.VMEM((EPW,), jnp.float32),
            pltpu.VMEM((TSTRIDE * LANES,), jnp.float32),
            pltpu.SemaphoreType.DMA,
            pltpu.SemaphoreType.DMA,
            pltpu.SemaphoreType.DMA,
            pltpu.SemaphoreType.DMA,
        ],
        compiler_params=pltpu.CompilerParams(needs_layout_passes=False),
    )


@jax.jit
def _classifier(x_req, x_code, src, dst):
    return _make_classifier()(x_req, x_code, src, dst)


def kernel(x_req, x_code, edge_label_index):
    src = edge_label_index[0].astype(jnp.int32)
    dst = edge_label_index[1].astype(jnp.int32)
    return _classifier(x_req, x_code, src, dst)


# speedup vs baseline: 1.0337x; 1.0337x over previous
import jax
import jax.numpy as jnp
from jax import lax
from jax.experimental import pallas as pl
from jax.experimental.pallas import tpu as pltpu
from jax.experimental.pallas import tpu_sc as plsc

N_NODES = 10000
D = 128
N_EDGES = 320000

NC = 2
NS = 16
NW = NC * NS
EPW = N_EDGES // NW
CHUNK = 80
NCHUNKS = EPW // CHUNK
LANES = 16
GROUPS = CHUNK // LANES
NBUF = 4


def _body(x_req_hbm, x_code_hbm, src_hbm, dst_hbm, out_hbm,
          src_v, dst_v,
          req0, req1, req2, req3, code0, code1, code2, code3,
          out_v, sem0, sem1, sem2, sem3):
    reqs = (req0, req1, req2, req3)
    codes = (code0, code1, code2, code3)
    sems = (sem0, sem1, sem2, sem3)

    wid = lax.axis_index("s") * NC + lax.axis_index("c")
    wbase = wid * EPW

    pltpu.sync_copy(src_hbm.at[pl.ds(wbase, EPW)], src_v)
    pltpu.sync_copy(dst_hbm.at[pl.ds(wbase, EPW)], dst_v)

    def fire(c, b):
        off = c * CHUNK
        pltpu.async_copy(x_req_hbm.at[src_v.at[pl.ds(off, CHUNK)]],
                         reqs[b], sems[b])
        pltpu.async_copy(x_code_hbm.at[dst_v.at[pl.ds(off, CHUNK)]],
                         codes[b], sems[b])

    def wait_rows(b):
        pltpu.make_async_copy(x_req_hbm.at[pl.ds(0, CHUNK)], reqs[b],
                              sems[b]).wait()
        pltpu.make_async_copy(x_code_hbm.at[pl.ds(0, CHUNK)], codes[b],
                              sems[b]).wait()

    lane_ids = lax.iota(jnp.int32, LANES)

    def compute(c, b):
        req_v, code_v = reqs[b], codes[b]
        obase = c * CHUNK

        @pl.loop(0, GROUPS)
        def _(g):
            gbase = g * LANES

            def edge_body(e, acc):
                edge = gbase + e
                ps = [req_v[edge, pl.ds(k * LANES, LANES)]
                      * code_v[edge, pl.ds(k * LANES, LANES)]
                      for k in range(D // LANES)]
                while len(ps) > 1:
                    ps = [ps[j] + ps[j + 1] for j in range(0, len(ps), 2)]
                s = jnp.sum(ps[0])
                return jnp.where(lane_ids == e, s, acc)

            acc = pl.loop(0, LANES, init_carry=jnp.zeros((LANES,), jnp.float32),
                          unroll=2)(edge_body)
            sig = 1.0 / (1.0 + jnp.exp(-acc))
            out_v[pl.ds(obase + gbase, LANES)] = sig

    for b in range(NBUF - 1):
        fire(b, b)

    @pl.loop(0, NCHUNKS - 1, step=NBUF)
    def _(i):
        for b in range(NBUF):
            c = i + b
            wait_rows(b)
            nxt = c + NBUF - 1

            @pl.when(nxt < NCHUNKS)
            def _():
                fire(nxt, (b + NBUF - 1) % NBUF)

            compute(c, b)

    wait_rows(0)
    compute(NCHUNKS - 1, 0)

    pltpu.sync_copy(out_v, out_hbm.at[pl.ds(wbase, EPW)])


def _make_classifier():
    mesh = plsc.VectorSubcoreMesh(core_axis_name="c", subcore_axis_name="s",
                                  num_cores=NC, num_subcores=NS)
    row_t = pltpu.VMEM((CHUNK, D), jnp.float32)
    return pl.kernel(
        _body,
        out_type=jax.ShapeDtypeStruct((N_EDGES,), jnp.float32),
        mesh=mesh,
        scratch_types=[
            pltpu.VMEM((EPW,), jnp.int32),
            pltpu.VMEM((EPW,), jnp.int32),
            row_t, row_t, row_t, row_t, row_t, row_t, row_t, row_t,
            pltpu.VMEM((EPW,), jnp.float32),
            pltpu.SemaphoreType.DMA,
            pltpu.SemaphoreType.DMA,
            pltpu.SemaphoreType.DMA,
            pltpu.SemaphoreType.DMA,
        ],
        compiler_params=pltpu.CompilerParams(needs_layout_passes=False),
    )


@jax.jit
def _classifier(x_req, x_code, src, dst):
    return _make_classifier()(x_req, x_code, src, dst)


def kernel(x_req, x_code, edge_label_index):
    src = edge_label_index[0].astype(jnp.int32)
    dst = edge_label_index[1].astype(jnp.int32)
    return _classifier(x_req, x_code, src, dst)
